# Initial kernel scaffold; baseline (speedup 1.0000x reference)
import jax
import jax.numpy as jnp
from jax.experimental import pallas as pl


def kernel(y_pred, y_true):
    raise NotImplementedError("write your pallas kernel here")


# speedup vs baseline: 15.7400x; 15.7400x over previous
import functools

import jax
import jax.numpy as jnp
from jax import lax
from jax.experimental import pallas as pl
from jax.experimental.pallas import tpu as pltpu
from jax.experimental.pallas import tpu_sc as plsc

N_REAL = 1_000_000
N_PAD = 1 << 20
K = N_REAL // 10
ALPHA = 5.0
NUM_WORKERS = 32
PER_TILE = N_PAD // NUM_WORKERS
VREGS_PER_TILE = PER_TILE // 16
NBINS = 4096
NSUB = 64
INT_MIN = -(2 ** 31)

_mesh = plsc.VectorSubcoreMesh(core_axis_name="c", subcore_axis_name="s")


def _keybits(v):
    bits = lax.bitcast_convert_type(v, jnp.int32)
    neg = lax.shift_right_arithmetic(bits, jnp.full((16,), 31, jnp.int32))
    return bits ^ (neg | jnp.full((16,), INT_MIN, jnp.int32))


def _bin12(key):
    return lax.shift_right_logical(key, jnp.full((16,), 20, jnp.int32))


@functools.partial(
    pl.kernel,
    out_type=jax.ShapeDtypeStruct((NUM_WORKERS, 32, 128), jnp.int32),
    mesh=_mesh,
    compiler_params=pltpu.CompilerParams(needs_layout_passes=False),
    scratch_types=[
        pltpu.VMEM((PER_TILE,), jnp.float32),
        pltpu.VMEM((16 * NBINS,), jnp.int32),
        pltpu.VMEM((32, 128), jnp.int32),
    ],
)
def _sc_hist(yt_hbm, out_hbm, yt_v, hist_v, merged_v):
    w = lax.axis_index("s") * 2 + lax.axis_index("c")
    zeros16 = jnp.zeros((16,), jnp.int32)

    def _zero(i, _):
        hist_v[pl.ds(i * 16, 16)] = zeros16
        return _

    lax.fori_loop(0, 16 * NBINS // 16, _zero, 0)

    pltpu.sync_copy(yt_hbm.at[pl.ds(w * PER_TILE, PER_TILE)], yt_v)

    lanebase = lax.iota(jnp.int32, 16) * NBINS
    ones16 = jnp.ones((16,), jnp.int32)

    def _accum(i, _):
        key = _keybits(yt_v[pl.ds(i * 16, 16)])
        plsc.addupdate_scatter(hist_v, [lanebase + _bin12(key)], ones16)
        return _

    lax.fori_loop(0, VREGS_PER_TILE, _accum, 0)

    def _merge(i, _):
        base = i * 16
        acc = hist_v[pl.ds(base, 16)]
        for s in range(1, 16):
            acc = acc + hist_v[pl.ds(s * NBINS + base, 16)]
        r = lax.shift_right_logical(i, 3)
        c = (i & 7) * 16
        merged_v[r, pl.ds(c, 16)] = acc
        return _

    lax.fori_loop(0, NBINS // 16, _merge, 0)
    pltpu.sync_copy(merged_v, out_hbm.at[w])


def _tc_scan_body(hist_ref, h_ref, meta_ref):
    h2 = jnp.sum(hist_ref[...], axis=0).astype(jnp.float32)
    iota_r = lax.broadcasted_iota(jnp.int32, (128, 128), 0)
    iota_c = lax.broadcasted_iota(jnp.int32, (128, 128), 1)
    suf_in_row = jnp.dot(h2, (iota_r >= iota_c).astype(jnp.float32),
                         preferred_element_type=jnp.float32)
    row_tot = suf_in_row[:, 0:1]
    i32r = lax.broadcasted_iota(jnp.int32, (32, 32), 0)
    i32c = lax.broadcasted_iota(jnp.int32, (32, 32), 1)
    strict_above = (i32c > i32r).astype(jnp.float32)
    row_suffix = jnp.dot(strict_above, row_tot,
                         preferred_element_type=jnp.float32)
    c_ge = row_suffix + suf_in_row
    kf = jnp.float32(K)
    h_bin = jnp.sum((c_ge >= kf).astype(jnp.int32)) - 1
    bin_id = (lax.broadcasted_iota(jnp.int32, (32, 128), 0) * 128
              + lax.broadcasted_iota(jnp.int32, (32, 128), 1))
    at_h = (bin_id == h_bin).astype(jnp.float32)
    cnt_h = jnp.sum(h2 * at_h)
    c_ge_h = jnp.sum(c_ge * at_h)
    c_gt = c_ge_h - cnt_h
    h_ref[...] = jnp.full((128,), h_bin, jnp.int32)
    meta_ref[...] = jnp.full((128,), c_gt, jnp.float32)


_tc_scan = pl.pallas_call(
    _tc_scan_body,
    out_shape=(jax.ShapeDtypeStruct((128,), jnp.int32),
               jax.ShapeDtypeStruct((128,), jnp.float32)),
)


@functools.partial(
    pl.kernel,
    out_type=(jax.ShapeDtypeStruct((NUM_WORKERS, 32), jnp.float32),
              jax.ShapeDtypeStruct((NUM_WORKERS, NSUB, 16), jnp.int32),
              jax.ShapeDtypeStruct((NUM_WORKERS, NSUB, 16), jnp.float32)),
    mesh=_mesh,
    compiler_params=pltpu.CompilerParams(needs_layout_passes=False),
    scratch_types=[
        pltpu.VMEM((PER_TILE,), jnp.float32),
        pltpu.VMEM((PER_TILE,), jnp.float32),
        pltpu.VMEM((16,), jnp.int32),
        pltpu.VMEM((32,), jnp.float32),
        pltpu.VMEM((NSUB, 16), jnp.int32),
        pltpu.VMEM((NSUB, 16), jnp.float32),
    ],
)
def _sc_sums(yp_hbm, yt_hbm, h_hbm, part_hbm, csub_hbm, esub_hbm,
             yp_v, yt_v, h_v, part_v, csub_v, esub_v):
    w = lax.axis_index("s") * 2 + lax.axis_index("c")
    zeros16i = jnp.zeros((16,), jnp.int32)
    zeros16f = jnp.zeros((16,), jnp.float32)

    def _zero(i, _):
        csub_v[i, pl.ds(0, 16)] = zeros16i
        esub_v[i, pl.ds(0, 16)] = zeros16f
        return _

    lax.fori_loop(0, NSUB, _zero, 0)

    pltpu.sync_copy(h_hbm.at[pl.ds(0, 16)], h_v)
    pltpu.sync_copy(yp_hbm.at[pl.ds(w * PER_TILE, PER_TILE)], yp_v)
    pltpu.sync_copy(yt_hbm.at[pl.ds(w * PER_TILE, PER_TILE)], yt_v)

    h_vec = h_v[pl.ds(0, 16)]
    lane = lax.iota(jnp.int32, 16)
    ones16 = jnp.ones((16,), jnp.int32)
    c14 = jnp.full((16,), 14, jnp.int32)
    m63 = jnp.full((16,), 63, jnp.int32)

    def _accum(i, carry):
        s_all, s_gt = carry
        t = yt_v[pl.ds(i * 16, 16)]
        p = yp_v[pl.ds(i * 16, 16)]
        d = t - p
        err = d * d
        key = _keybits(t)
        b = _bin12(key)
        s_all = s_all + err
        s_gt = s_gt + jnp.where(b > h_vec, err, 0.0)
        in_h = b == h_vec
        sub = lax.shift_right_logical(key, c14) & m63
        plsc.addupdate_scatter(csub_v, [sub, lane], ones16, mask=in_h)
        plsc.addupdate_scatter(esub_v, [sub, lane], err, mask=in_h)
        return s_all, s_gt

    s_all, s_gt = lax.fori_loop(0, VREGS_PER_TILE, _accum,
                                (zeros16f, zeros16f))
    part_v[pl.ds(0, 16)] = s_all
    part_v[pl.ds(16, 16)] = s_gt
    pltpu.sync_copy(part_v, part_hbm.at[w])
    pltpu.sync_copy(csub_v, csub_hbm.at[w])
    pltpu.sync_copy(esub_v, esub_hbm.at[w])


def _tc_final_body(part_ref, csub_ref, esub_ref, meta_ref, out_ref):
    part = part_ref[...]
    s_all = jnp.sum(part[:, 0:16])
    s_gt = jnp.sum(part[:, 16:32])
    csub = jnp.sum(csub_ref[...], axis=(0, 2)).astype(jnp.float32)
    esub = jnp.sum(esub_ref[...], axis=(0, 2))
    c_gt = meta_ref[0]
    m = jnp.float32(K) - c_gt

    cs2 = csub.reshape(1, NSUB)
    i64r = lax.broadcasted_iota(jnp.int32, (NSUB, NSUB), 0)
    i64c = lax.broadcasted_iota(jnp.int32, (NSUB, NSUB), 1)
    c_ge = jnp.dot(cs2, (i64r >= i64c).astype(jnp.float32),
                   preferred_element_type=jnp.float32)[0]
    hs = jnp.sum((c_ge >= m).astype(jnp.int32)) - 1
    sid = lax.iota(jnp.int32, NSUB)
    at_hs = (sid == hs).astype(jnp.float32)
    cnt_hs = jnp.sum(csub * at_hs)
    c_sub_gt = jnp.sum(c_ge * at_hs) - cnt_hs
    frac = (m - c_sub_gt) / jnp.maximum(cnt_hs, 1.0)
    e_above = jnp.sum(jnp.where(sid > hs, esub, 0.0))
    s_top = s_gt + e_above + frac * jnp.sum(esub * at_hs)
    loss = (s_all + jnp.float32(ALPHA - 1.0) * s_top) / jnp.float32(N_REAL)
    out_ref[...] = jnp.full((1, 1), loss, jnp.float32)


_tc_final = pl.pallas_call(
    _tc_final_body,
    out_shape=jax.ShapeDtypeStruct((1, 1), jnp.float32),
)


def kernel(y_pred, y_true):
    pad = jnp.zeros((N_PAD - N_REAL,), jnp.float32)
    yp = jnp.concatenate([y_pred, pad])
    yt = jnp.concatenate([y_true, pad])
    hist = _sc_hist(yt)
    h_splat, meta = _tc_scan(hist)
    part, csub, esub = _sc_sums(yp, yt, h_splat)
    loss = _tc_final(part, csub, esub, meta)
    return jnp.reshape(loss, ())


# speedup vs baseline: 38.0228x; 2.4157x over previous
import functools

import jax
import jax.numpy as jnp
from jax import lax
from jax.experimental import pallas as pl
from jax.experimental.pallas import tpu as pltpu
from jax.experimental.pallas import tpu_sc as plsc

N_REAL = 1_000_000
K = N_REAL // 10
ALPHA = 5.0
NUM_WORKERS = 32
PER_TILE = 31232
VREGS_PER_TILE = PER_TILE // 16
TAIL_START = NUM_WORKERS * PER_TILE
TAIL = N_REAL - TAIL_START
TAIL_VREGS = TAIL // 16
NBINS = 4096
NSUB = 64
INT_MIN = -(2 ** 31)

_mesh = plsc.VectorSubcoreMesh(core_axis_name="c", subcore_axis_name="s")


def _keybits(v):
    bits = lax.bitcast_convert_type(v, jnp.int32)
    neg = lax.shift_right_arithmetic(bits, jnp.full((16,), 31, jnp.int32))
    return bits ^ (neg | jnp.full((16,), INT_MIN, jnp.int32))


def _bin12(key):
    return lax.shift_right_logical(key, jnp.full((16,), 20, jnp.int32))


@functools.partial(
    pl.kernel,
    out_type=jax.ShapeDtypeStruct((NUM_WORKERS, 32, 128), jnp.int32),
    mesh=_mesh,
    compiler_params=pltpu.CompilerParams(needs_layout_passes=False),
    scratch_types=[
        pltpu.VMEM((PER_TILE,), jnp.float32),
        pltpu.VMEM((TAIL,), jnp.float32),
        pltpu.VMEM((32, 128), jnp.int32),
    ],
)
def _sc_hist(yt_hbm, out_hbm, yt_v, tail_v, hist_v):
    w = lax.axis_index("s") * 2 + lax.axis_index("c")
    zeros16 = jnp.zeros((16,), jnp.int32)

    @plsc.parallel_loop(0, NBINS // 16, step=8)
    def _zero(i):
        base = i * 16
        for u in range(8):
            j = base + u * 16
            hist_v[lax.shift_right_logical(j, 7), pl.ds(j & 127, 16)] = zeros16

    pltpu.sync_copy(yt_hbm.at[pl.ds(w * PER_TILE, PER_TILE)], yt_v)

    ones16 = jnp.ones((16,), jnp.int32)
    c7 = jnp.full((16,), 7, jnp.int32)
    m127 = jnp.full((16,), 127, jnp.int32)

    @plsc.parallel_loop(0, VREGS_PER_TILE, step=8)
    def _accum(i):
        for u in range(8):
            b = _bin12(_keybits(yt_v[pl.ds((i + u) * 16, 16)]))
            r = lax.shift_right_logical(b, c7)
            plsc.addupdate_scatter(hist_v, [r, b & m127], ones16)

    pltpu.sync_copy(yt_hbm.at[pl.ds(TAIL_START, TAIL)], tail_v)
    is31 = jnp.full((16,), w, jnp.int32) == jnp.full((16,), 31, jnp.int32)

    @plsc.parallel_loop(0, TAIL_VREGS, step=4)
    def _tail(i):
        for u in range(4):
            b = _bin12(_keybits(tail_v[pl.ds((i + u) * 16, 16)]))
            r = lax.shift_right_logical(b, c7)
            plsc.addupdate_scatter(hist_v, [r, b & m127], ones16, mask=is31)

    pltpu.sync_copy(hist_v, out_hbm.at[w])


def _tc_scan_body(hist_ref, h_ref, meta_ref):
    h2 = jnp.sum(hist_ref[...], axis=0).astype(jnp.float32)
    iota_r = lax.broadcasted_iota(jnp.int32, (128, 128), 0)
    iota_c = lax.broadcasted_iota(jnp.int32, (128, 128), 1)
    suf_in_row = jnp.dot(h2, (iota_r >= iota_c).astype(jnp.float32),
                         preferred_element_type=jnp.float32)
    row_tot = suf_in_row[:, 0:1]
    i32r = lax.broadcasted_iota(jnp.int32, (32, 32), 0)
    i32c = lax.broadcasted_iota(jnp.int32, (32, 32), 1)
    strict_above = (i32c > i32r).astype(jnp.float32)
    row_suffix = jnp.dot(strict_above, row_tot,
                         preferred_element_type=jnp.float32)
    c_ge = row_suffix + suf_in_row
    kf = jnp.float32(K)
    h_bin = jnp.sum((c_ge >= kf).astype(jnp.int32)) - 1
    bin_id = (lax.broadcasted_iota(jnp.int32, (32, 128), 0) * 128
              + lax.broadcasted_iota(jnp.int32, (32, 128), 1))
    at_h = (bin_id == h_bin).astype(jnp.float32)
    cnt_h = jnp.sum(h2 * at_h)
    c_ge_h = jnp.sum(c_ge * at_h)
    c_gt = c_ge_h - cnt_h
    h_ref[...] = jnp.full((128,), h_bin, jnp.int32)
    meta_ref[...] = jnp.full((128,), c_gt, jnp.float32)


_tc_scan = pl.pallas_call(
    _tc_scan_body,
    out_shape=(jax.ShapeDtypeStruct((128,), jnp.int32),
               jax.ShapeDtypeStruct((128,), jnp.float32)),
)


@functools.partial(
    pl.kernel,
    out_type=(jax.ShapeDtypeStruct((NUM_WORKERS, 32), jnp.float32),
              jax.ShapeDtypeStruct((NUM_WORKERS, NSUB, 16), jnp.int32),
              jax.ShapeDtypeStruct((NUM_WORKERS, NSUB, 16), jnp.float32)),
    mesh=_mesh,
    compiler_params=pltpu.CompilerParams(needs_layout_passes=False),
    scratch_types=[
        pltpu.VMEM((PER_TILE,), jnp.float32),
        pltpu.VMEM((PER_TILE,), jnp.float32),
        pltpu.VMEM((TAIL,), jnp.float32),
        pltpu.VMEM((TAIL,), jnp.float32),
        pltpu.VMEM((16,), jnp.int32),
        pltpu.VMEM((32,), jnp.float32),
        pltpu.VMEM((NSUB, 16), jnp.int32),
        pltpu.VMEM((NSUB, 16), jnp.float32),
    ],
)
def _sc_sums(yp_hbm, yt_hbm, h_hbm, part_hbm, csub_hbm, esub_hbm,
             yp_v, yt_v, tp_v, tt_v, h_v, part_v, csub_v, esub_v):
    w = lax.axis_index("s") * 2 + lax.axis_index("c")
    zeros16i = jnp.zeros((16,), jnp.int32)
    zeros16f = jnp.zeros((16,), jnp.float32)

    def _zero(i, _):
        csub_v[i, pl.ds(0, 16)] = zeros16i
        esub_v[i, pl.ds(0, 16)] = zeros16f
        return _

    lax.fori_loop(0, NSUB, _zero, 0)

    pltpu.sync_copy(h_hbm.at[pl.ds(0, 16)], h_v)
    pltpu.sync_copy(yp_hbm.at[pl.ds(w * PER_TILE, PER_TILE)], yp_v)
    pltpu.sync_copy(yt_hbm.at[pl.ds(w * PER_TILE, PER_TILE)], yt_v)
    pltpu.sync_copy(yp_hbm.at[pl.ds(TAIL_START, TAIL)], tp_v)
    pltpu.sync_copy(yt_hbm.at[pl.ds(TAIL_START, TAIL)], tt_v)

    h_vec = h_v[pl.ds(0, 16)]
    lane = lax.iota(jnp.int32, 16)
    ones16 = jnp.ones((16,), jnp.int32)
    c14 = jnp.full((16,), 14, jnp.int32)
    m63 = jnp.full((16,), 63, jnp.int32)

    def _one(t, p, gate=None):
        d = t - p
        err = d * d
        key = _keybits(t)
        b = _bin12(key)
        if gate is not None:
            err = jnp.where(gate, err, 0.0)
        e_gt = jnp.where(b > h_vec, err, 0.0)
        in_h = b == h_vec
        if gate is not None:
            in_h = jnp.logical_and(in_h, gate)
        sub = lax.shift_right_logical(key, c14) & m63
        plsc.addupdate_scatter(csub_v, [sub, lane], ones16, mask=in_h)
        plsc.addupdate_scatter(esub_v, [sub, lane], err, mask=in_h)
        return err, e_gt

    @plsc.parallel_loop(0, VREGS_PER_TILE, step=8,
                        carry=(zeros16f, zeros16f))
    def _accum(i, carry):
        s_all, s_gt = carry
        ea, eg = [], []
        for u in range(8):
            err, e_gt = _one(yt_v[pl.ds((i + u) * 16, 16)],
                             yp_v[pl.ds((i + u) * 16, 16)])
            ea.append(err)
            eg.append(e_gt)
        for lst in (ea, eg):
            for u in (0, 2, 4, 6):
                lst[u] = lst[u] + lst[u + 1]
            lst[0] = (lst[0] + lst[2]) + (lst[4] + lst[6])
        return s_all + ea[0], s_gt + eg[0]

    is31 = jnp.full((16,), w, jnp.int32) == jnp.full((16,), 31, jnp.int32)

    @plsc.parallel_loop(0, TAIL_VREGS, step=4, carry=_accum)
    def _tail(i, carry):
        s_all, s_gt = carry
        for u in range(4):
            err, e_gt = _one(tt_v[pl.ds((i + u) * 16, 16)],
                             tp_v[pl.ds((i + u) * 16, 16)], gate=is31)
            s_all = s_all + err
            s_gt = s_gt + e_gt
        return s_all, s_gt

    s_all, s_gt = _tail
    part_v[pl.ds(0, 16)] = s_all
    part_v[pl.ds(16, 16)] = s_gt
    pltpu.sync_copy(part_v, part_hbm.at[w])
    pltpu.sync_copy(csub_v, csub_hbm.at[w])
    pltpu.sync_copy(esub_v, esub_hbm.at[w])


def _tc_final_body(part_ref, csub_ref, esub_ref, meta_ref, out_ref):
    part = part_ref[...]
    s_all = jnp.sum(part[:, 0:16])
    s_gt = jnp.sum(part[:, 16:32])
    csub = jnp.sum(csub_ref[...], axis=(0, 2)).astype(jnp.float32)
    esub = jnp.sum(esub_ref[...], axis=(0, 2))
    c_gt = meta_ref[0]
    m = jnp.float32(K) - c_gt

    cs2 = csub.reshape(1, NSUB)
    i64r = lax.broadcasted_iota(jnp.int32, (NSUB, NSUB), 0)
    i64c = lax.broadcasted_iota(jnp.int32, (NSUB, NSUB), 1)
    c_ge = jnp.dot(cs2, (i64r >= i64c).astype(jnp.float32),
                   preferred_element_type=jnp.float32)[0]
    hs = jnp.sum((c_ge >= m).astype(jnp.int32)) - 1
    sid = lax.iota(jnp.int32, NSUB)
    at_hs = (sid == hs).astype(jnp.float32)
    cnt_hs = jnp.sum(csub * at_hs)
    c_sub_gt = jnp.sum(c_ge * at_hs) - cnt_hs
    frac = (m - c_sub_gt) / jnp.maximum(cnt_hs, 1.0)
    e_above = jnp.sum(jnp.where(sid > hs, esub, 0.0))
    s_top = s_gt + e_above + frac * jnp.sum(esub * at_hs)
    loss = (s_all + jnp.float32(ALPHA - 1.0) * s_top) / jnp.float32(N_REAL)
    out_ref[...] = jnp.full((1, 1), loss, jnp.float32)


_tc_final = pl.pallas_call(
    _tc_final_body,
    out_shape=jax.ShapeDtypeStruct((1, 1), jnp.float32),
)


def kernel(y_pred, y_true):
    hist = _sc_hist(y_true)
    h_splat, meta = _tc_scan(hist)
    part, csub, esub = _sc_sums(y_pred, y_true, h_splat)
    loss = _tc_final(part, csub, esub, meta)
    return jnp.reshape(loss, ())
